# baseline (speedup 1.0000x reference)
import functools

import jax
import jax.numpy as jnp
from jax import lax
from jax.experimental import pallas as pl
from jax.experimental.pallas import tpu as pltpu
from jax.experimental.pallas import tpu_sc as plsc

N = 10000
E = 320000
H = 128
C = 128
NW = 32
NSLAB = 4
ES = E // NSLAB
SCHUNKS = ES // C
KBASE = SCHUNKS // NW
KREM = SCHUNKS % NW
STRIPE = 624
TAIL0 = 16 * STRIPE
TAILN = N - TAIL0

f32 = jnp.float32


def _ln(y, g, b):
    m = jnp.mean(y, axis=-1, keepdims=True)
    v = jnp.mean((y - m) ** 2, axis=-1, keepdims=True)
    return (y - m) * lax.rsqrt(v + 1e-5) * g + b


def _sc_mesh():
    return plsc.VectorSubcoreMesh(core_axis_name="c", subcore_axis_name="s")


def _gather_add(ps, pd, src, dst, slab):
    @functools.partial(
        pl.kernel,
        out_type=jax.ShapeDtypeStruct((ES, H), f32),
        mesh=_sc_mesh(),
        scratch_types=[
            pltpu.VMEM((2, C), jnp.int32),
            pltpu.VMEM((2, C), jnp.int32),
            pltpu.VMEM((2, C, H), f32),
            pltpu.VMEM((2, C, H), f32),
            pltpu.SemaphoreType.DMA,
            pltpu.SemaphoreType.DMA,
            pltpu.SemaphoreType.DMA,
            pltpu.SemaphoreType.DMA,
        ],
    )
    def k(ps_h, pd_h, src_h, dst_h, g_h, idx_s, idx_d, buf_a, buf_b,
          g0, g1, w0, w1):
        cid = lax.axis_index("c")
        sid = lax.axis_index("s")
        wid = cid * 16 + sid
        nk = KBASE + (wid < KREM).astype(jnp.int32)
        gsem = (g0, g1)
        wsem = (w0, w1)

        def start_gathers(slot, i):
            base = (slab * SCHUNKS + wid + i * NW) * C
            pltpu.sync_copy(src_h.at[pl.ds(base, C)], idx_s.at[slot])
            pltpu.sync_copy(dst_h.at[pl.ds(base, C)], idx_d.at[slot])
            pltpu.async_copy(ps_h.at[idx_s.at[slot]], buf_a.at[slot], gsem[slot])
            pltpu.async_copy(pd_h.at[idx_d.at[slot]], buf_b.at[slot], gsem[slot])

        def wait_gathers(slot):
            pltpu.make_async_copy(ps_h.at[idx_s.at[slot]], buf_a.at[slot], gsem[slot]).wait()
            pltpu.make_async_copy(pd_h.at[idx_d.at[slot]], buf_b.at[slot], gsem[slot]).wait()

        def wait_wb(slot):
            pltpu.make_async_copy(buf_a.at[slot], g_h.at[pl.ds(0, C)], wsem[slot]).wait()

        start_gathers(0, 0)

        def body(i, carry):
            def step(cur, nxt):
                @pl.when(i + 1 < nk)
                def _():
                    @pl.when(i >= 1)
                    def _():
                        wait_wb(nxt)

                    start_gathers(nxt, i + 1)

                wait_gathers(cur)

                def add_row(j, c2):
                    for t in range(H // 16):
                        sl = pl.ds(t * 16, 16)
                        buf_a[cur, j, sl] = buf_a[cur, j, sl] + buf_b[cur, j, sl]
                    return c2

                lax.fori_loop(0, C, add_row, 0)
                base = (wid + i * NW) * C
                pltpu.async_copy(buf_a.at[cur], g_h.at[pl.ds(base, C)], wsem[cur])

            @pl.when(lax.rem(i, 2) == 0)
            def _():
                step(0, 1)

            @pl.when(lax.rem(i, 2) == 1)
            def _():
                step(1, 0)

            return carry

        lax.fori_loop(0, nk, body, 0)
        wait_wb(0)
        wait_wb(1)

    return k(ps, pd, src, dst)


def _seg_sum(e, dst, zeros_nh, slab):
    @functools.partial(
        pl.kernel,
        out_type=jax.ShapeDtypeStruct((2, N, H), f32),
        mesh=_sc_mesh(),
        scratch_types=[
            pltpu.VMEM((2, C), jnp.int32),
            pltpu.VMEM((2, C, H), f32),
            pltpu.VMEM_SHARED((N, H), f32),
            pltpu.SemaphoreType.DMA,
            pltpu.SemaphoreType.DMA,
            pltpu.SemaphoreType.DMA,
            pltpu.SemaphoreType.DMA,
        ],
    )
    def k(e_h, dst_h, z_h, out_h, idx, buf, acc, l0, l1, s0, s1):
        cid = lax.axis_index("c")
        sid = lax.axis_index("s")
        wid = cid * 16 + sid
        nk = KBASE + (wid < KREM).astype(jnp.int32)
        lsem = (l0, l1)
        ssem = (s0, s1)

        def start_loads(slot, i):
            lbase = (wid + i * NW) * C
            pltpu.async_copy(dst_h.at[pl.ds(slab * ES + lbase, C)], idx.at[slot], lsem[slot])
            pltpu.async_copy(e_h.at[pl.ds(lbase, C)], buf.at[slot], lsem[slot])

        def wait_loads(slot):
            pltpu.make_async_copy(dst_h.at[pl.ds(0, C)], idx.at[slot], lsem[slot]).wait()
            pltpu.make_async_copy(e_h.at[pl.ds(0, C)], buf.at[slot], lsem[slot]).wait()

        def wait_scatter(slot):
            pltpu.make_async_copy(buf.at[slot], acc.at[idx.at[slot]], ssem[slot]).wait()

        start_loads(0, 0)
        r0 = sid * STRIPE
        pltpu.sync_copy(z_h.at[pl.ds(r0, STRIPE)], acc.at[pl.ds(r0, STRIPE)])

        @pl.when(sid == 15)
        def _():
            pltpu.sync_copy(z_h.at[pl.ds(TAIL0, TAILN)], acc.at[pl.ds(TAIL0, TAILN)])

        plsc.subcore_barrier()

        def body(i, carry):
            def step(cur, nxt):
                wait_loads(cur)

                @pl.when(i + 1 < nk)
                def _():
                    @pl.when(i >= 1)
                    def _():
                        wait_scatter(nxt)

                    start_loads(nxt, i + 1)

                pltpu.async_copy(buf.at[cur], acc.at[idx.at[cur]], ssem[cur],
                                 add=True)

            @pl.when(lax.rem(i, 2) == 0)
            def _():
                step(0, 1)

            @pl.when(lax.rem(i, 2) == 1)
            def _():
                step(1, 0)

            return carry

        lax.fori_loop(0, nk, body, 0)
        wait_scatter(0)
        wait_scatter(1)
        plsc.subcore_barrier()
        pltpu.sync_copy(acc.at[pl.ds(r0, STRIPE)],
                        out_h.at[cid, pl.ds(r0, STRIPE)])

        @pl.when(sid == 15)
        def _():
            pltpu.sync_copy(acc.at[pl.ds(TAIL0, TAILN)],
                            out_h.at[cid, pl.ds(TAIL0, TAILN)])

    return k(e, dst, zeros_nh)


_NB = 1000
_EB = 2000


def _w_spec(shape):
    nd = len(shape)
    return pl.BlockSpec(shape, (lambda i: (0,) * nd))


def _node_encode(x, W1, b1, W2, b2, g, be, Ws, Wd):
    def body(x_r, W1_r, b1_r, W2_r, b2_r, g_r, be_r, Ws_r, Wd_r, h_r, ps_r, pd_r):
        u = jax.nn.silu(jnp.dot(x_r[...], W1_r[...]) + b1_r[...])
        y = jnp.dot(u, W2_r[...]) + b2_r[...]
        h = _ln(y, g_r[...], be_r[...])
        h_r[...] = h
        ps_r[...] = jnp.dot(h, Ws_r[...])
        pd_r[...] = jnp.dot(h, Wd_r[...])

    nblk = pl.BlockSpec((_NB, H), lambda i: (i, 0))
    return pl.pallas_call(
        body,
        grid=(N // _NB,),
        in_specs=[pl.BlockSpec((_NB, x.shape[1]), lambda i: (i, 0)),
                  _w_spec(W1.shape), _w_spec(b1.shape), _w_spec(W2.shape),
                  _w_spec(b2.shape), _w_spec(g.shape), _w_spec(be.shape),
                  _w_spec(Ws.shape), _w_spec(Wd.shape)],
        out_specs=[nblk, nblk, nblk],
        out_shape=[jax.ShapeDtypeStruct((N, H), f32)] * 3,
    )(x, W1, b1, W2, b2, g, be, Ws, Wd)


def _edge_encode(ea, W1, b1, W2, b2, g, be, slab):
    def body(ea_r, W1_r, b1_r, W2_r, b2_r, g_r, be_r, e_r):
        u = jax.nn.silu(jnp.dot(ea_r[...], W1_r[...]) + b1_r[...])
        y = jnp.dot(u, W2_r[...]) + b2_r[...]
        e_r[...] = _ln(y, g_r[...], be_r[...])

    off = slab * (ES // _EB)
    return pl.pallas_call(
        body,
        grid=(ES // _EB,),
        in_specs=[pl.BlockSpec((_EB, ea.shape[1]), lambda i: (i + off, 0)),
                  _w_spec(W1.shape), _w_spec(b1.shape), _w_spec(W2.shape),
                  _w_spec(b2.shape), _w_spec(g.shape), _w_spec(be.shape)],
        out_specs=pl.BlockSpec((_EB, H), lambda i: (i, 0)),
        out_shape=jax.ShapeDtypeStruct((ES, H), f32),
    )(ea, W1, b1, W2, b2, g, be)


def _edge_mlp(e, gsum, We, b1, W2, b2, g, be):
    def body(e_r, gs_r, We_r, b1_r, W2_r, b2_r, g_r, be_r, o_r):
        e0 = e_r[...]
        u = jax.nn.silu(jnp.dot(e0, We_r[...]) + gs_r[...] + b1_r[...])
        y = jnp.dot(u, W2_r[...]) + b2_r[...]
        o_r[...] = e0 + _ln(y, g_r[...], be_r[...])

    eblk = pl.BlockSpec((_EB, H), lambda i: (i, 0))
    return pl.pallas_call(
        body,
        grid=(ES // _EB,),
        in_specs=[eblk, eblk, _w_spec(We.shape), _w_spec(b1.shape),
                  _w_spec(W2.shape), _w_spec(b2.shape), _w_spec(g.shape),
                  _w_spec(be.shape)],
        out_specs=eblk,
        out_shape=jax.ShapeDtypeStruct((ES, H), f32),
    )(e, gsum, We, b1, W2, b2, g, be)


def _node_update(h, parts_list, Wh, Wa, b1, W2, b2, g, be, Ws, Wd):
    def body(h_r, *refs):
        (p_refs, (Wh_r, Wa_r, b1_r, W2_r, b2_r, g_r, be_r, Ws_r, Wd_r),
         (hn_r, ps_r, pd_r)) = refs[:NSLAB], refs[NSLAB:NSLAB + 9], refs[NSLAB + 9:]
        h0 = h_r[...]
        agg = sum(p_r[0] + p_r[1] for p_r in p_refs)
        u = jax.nn.silu(jnp.dot(h0, Wh_r[...]) + jnp.dot(agg, Wa_r[...]) + b1_r[...])
        y = jnp.dot(u, W2_r[...]) + b2_r[...]
        hn = h0 + _ln(y, g_r[...], be_r[...])
        hn_r[...] = hn
        ps_r[...] = jnp.dot(hn, Ws_r[...])
        pd_r[...] = jnp.dot(hn, Wd_r[...])

    nblk = pl.BlockSpec((_NB, H), lambda i: (i, 0))
    pblk = pl.BlockSpec((2, _NB, H), lambda i: (0, i, 0))
    return pl.pallas_call(
        body,
        grid=(N // _NB,),
        in_specs=[nblk] + [pblk] * NSLAB +
                 [_w_spec(Wh.shape), _w_spec(Wa.shape), _w_spec(b1.shape),
                  _w_spec(W2.shape), _w_spec(b2.shape), _w_spec(g.shape),
                  _w_spec(be.shape), _w_spec(Ws.shape), _w_spec(Wd.shape)],
        out_specs=[nblk, nblk, nblk],
        out_shape=[jax.ShapeDtypeStruct((N, H), f32)] * 3,
    )(h, *parts_list, Wh, Wa, b1, W2, b2, g, be, Ws, Wd)


def _node_update_final(h, parts_list, Wh, Wa, b1, W2, b2, g, be, dW1, db1, dW2, db2):
    def body(h_r, *refs):
        (p_refs, (Wh_r, Wa_r, b1_r, W2_r, b2_r, g_r, be_r,
                  dW1_r, db1_r, dW2_r, db2_r), (o_r,)) = (
            refs[:NSLAB], refs[NSLAB:NSLAB + 11], refs[NSLAB + 11:])
        h0 = h_r[...]
        agg = sum(p_r[0] + p_r[1] for p_r in p_refs)
        u = jax.nn.silu(jnp.dot(h0, Wh_r[...]) + jnp.dot(agg, Wa_r[...]) + b1_r[...])
        y = jnp.dot(u, W2_r[...]) + b2_r[...]
        hn = h0 + _ln(y, g_r[...], be_r[...])
        d = jax.nn.silu(jnp.dot(hn, dW1_r[...]) + db1_r[...])
        o_r[...] = jnp.dot(d, dW2_r[...]) + db2_r[...]

    nblk = pl.BlockSpec((_NB, H), lambda i: (i, 0))
    pblk = pl.BlockSpec((2, _NB, H), lambda i: (0, i, 0))
    D_OUT = dW2.shape[1]
    return pl.pallas_call(
        body,
        grid=(N // _NB,),
        in_specs=[nblk] + [pblk] * NSLAB +
                 [_w_spec(Wh.shape), _w_spec(Wa.shape), _w_spec(b1.shape),
                  _w_spec(W2.shape), _w_spec(b2.shape), _w_spec(g.shape),
                  _w_spec(be.shape), _w_spec(dW1.shape), _w_spec(db1.shape),
                  _w_spec(dW2.shape), _w_spec(db2.shape)],
        out_specs=pl.BlockSpec((_NB, D_OUT), lambda i: (i, 0)),
        out_shape=jax.ShapeDtypeStruct((N, D_OUT), f32),
    )(h, *parts_list, Wh, Wa, b1, W2, b2, g, be, dW1, db1, dW2, db2)


def kernel(x, edge_index, edge_attr, en_W1, en_b1, en_W2, en_b2, en_g, en_be, ee_W1, ee_b1, ee_W2, ee_b2, ee_g, ee_be, pe_W1, pe_b1, pe_W2, pe_b2, pe_g, pe_be, pn_W1, pn_b1, pn_W2, pn_b2, pn_g, pn_be, dec_W1, dec_b1, dec_W2, dec_b2):
    src = edge_index[0]
    dst = edge_index[1]
    L = pe_W1.shape[0]
    zeros_nh = jnp.zeros((N, H), f32)

    def row(v):
        return v.reshape(1, -1)

    h, ps, pd = _node_encode(x, en_W1, row(en_b1), en_W2, row(en_b2),
                             row(en_g), row(en_be),
                             pe_W1[0, H:2 * H], pe_W1[0, 2 * H:])
    e = [_edge_encode(edge_attr, ee_W1, row(ee_b1), ee_W2, row(ee_b2),
                      row(ee_g), row(ee_be), s_)
         for s_ in range(NSLAB)]

    for l in range(L):
        parts = [None] * NSLAB
        for s_ in range(NSLAB):
            g = _gather_add(ps, pd, src, dst, s_)
            e[s_] = _edge_mlp(e[s_], g, pe_W1[l, :H], row(pe_b1[l]), pe_W2[l],
                              row(pe_b2[l]), row(pe_g[l]), row(pe_be[l]))
            parts[s_] = _seg_sum(e[s_], dst, zeros_nh, s_)
        if l < L - 1:
            h, ps, pd = _node_update(
                h, parts, pn_W1[l, :H], pn_W1[l, H:], row(pn_b1[l]),
                pn_W2[l], row(pn_b2[l]), row(pn_g[l]), row(pn_be[l]),
                pe_W1[l + 1, H:2 * H], pe_W1[l + 1, 2 * H:])
        else:
            out = _node_update_final(
                h, parts, pn_W1[l, :H], pn_W1[l, H:], row(pn_b1[l]),
                pn_W2[l], row(pn_b2[l]), row(pn_g[l]), row(pn_be[l]),
                dec_W1, row(dec_b1), dec_W2, row(dec_b2))
    return out


# speedup vs baseline: 1.0270x; 1.0270x over previous
import functools

import jax
import jax.numpy as jnp
from jax import lax
from jax.experimental import pallas as pl
from jax.experimental.pallas import tpu as pltpu
from jax.experimental.pallas import tpu_sc as plsc

N = 10000
E = 320000
H = 128
C = 128
NW = 32
NSLAB = 2
ES = E // NSLAB
SCHUNKS = ES // C
KBASE = SCHUNKS // NW
KREM = SCHUNKS % NW
STRIPE = 624
TAIL0 = 16 * STRIPE
TAILN = N - TAIL0

f32 = jnp.float32


def _ln(y, g, b):
    m = jnp.mean(y, axis=-1, keepdims=True)
    v = jnp.mean((y - m) ** 2, axis=-1, keepdims=True)
    return (y - m) * lax.rsqrt(v + 1e-5) * g + b


def _sc_mesh():
    return plsc.VectorSubcoreMesh(core_axis_name="c", subcore_axis_name="s")


def _gather_add(ps, pd, src, dst, slab):
    @functools.partial(
        pl.kernel,
        out_type=jax.ShapeDtypeStruct((ES, H), f32),
        mesh=_sc_mesh(),
        scratch_types=[
            pltpu.VMEM((2, C), jnp.int32),
            pltpu.VMEM((2, C), jnp.int32),
            pltpu.VMEM((2, C, H), f32),
            pltpu.VMEM((2, C, H), f32),
            pltpu.SemaphoreType.DMA,
            pltpu.SemaphoreType.DMA,
            pltpu.SemaphoreType.DMA,
            pltpu.SemaphoreType.DMA,
        ],
    )
    def k(ps_h, pd_h, src_h, dst_h, g_h, idx_s, idx_d, buf_a, buf_b,
          g0, g1, w0, w1):
        cid = lax.axis_index("c")
        sid = lax.axis_index("s")
        wid = cid * 16 + sid
        nk = KBASE + (wid < KREM).astype(jnp.int32)
        gsem = (g0, g1)
        wsem = (w0, w1)

        def start_gathers(slot, i):
            base = (slab * SCHUNKS + wid + i * NW) * C
            pltpu.sync_copy(src_h.at[pl.ds(base, C)], idx_s.at[slot])
            pltpu.sync_copy(dst_h.at[pl.ds(base, C)], idx_d.at[slot])
            pltpu.async_copy(ps_h.at[idx_s.at[slot]], buf_a.at[slot], gsem[slot])
            pltpu.async_copy(pd_h.at[idx_d.at[slot]], buf_b.at[slot], gsem[slot])

        def wait_gathers(slot):
            pltpu.make_async_copy(ps_h.at[idx_s.at[slot]], buf_a.at[slot], gsem[slot]).wait()
            pltpu.make_async_copy(pd_h.at[idx_d.at[slot]], buf_b.at[slot], gsem[slot]).wait()

        def wait_wb(slot):
            pltpu.make_async_copy(buf_a.at[slot], g_h.at[pl.ds(0, C)], wsem[slot]).wait()

        start_gathers(0, 0)

        def body(i, carry):
            def step(cur, nxt):
                @pl.when(i + 1 < nk)
                def _():
                    @pl.when(i >= 1)
                    def _():
                        wait_wb(nxt)

                    start_gathers(nxt, i + 1)

                wait_gathers(cur)

                def add_row(j, c2):
                    for t in range(H // 16):
                        sl = pl.ds(t * 16, 16)
                        buf_a[cur, j, sl] = buf_a[cur, j, sl] + buf_b[cur, j, sl]
                    return c2

                lax.fori_loop(0, C, add_row, 0)
                base = (wid + i * NW) * C
                pltpu.async_copy(buf_a.at[cur], g_h.at[pl.ds(base, C)], wsem[cur])

            @pl.when(lax.rem(i, 2) == 0)
            def _():
                step(0, 1)

            @pl.when(lax.rem(i, 2) == 1)
            def _():
                step(1, 0)

            return carry

        lax.fori_loop(0, nk, body, 0)
        wait_wb(0)
        wait_wb(1)

    return k(ps, pd, src, dst)


def _seg_sum(e, dst, zeros_nh, slab):
    @functools.partial(
        pl.kernel,
        out_type=jax.ShapeDtypeStruct((2, N, H), f32),
        mesh=_sc_mesh(),
        scratch_types=[
            pltpu.VMEM((2, C), jnp.int32),
            pltpu.VMEM((2, C, H), f32),
            pltpu.VMEM_SHARED((N, H), f32),
            pltpu.SemaphoreType.DMA,
            pltpu.SemaphoreType.DMA,
            pltpu.SemaphoreType.DMA,
            pltpu.SemaphoreType.DMA,
        ],
    )
    def k(e_h, dst_h, z_h, out_h, idx, buf, acc, l0, l1, s0, s1):
        cid = lax.axis_index("c")
        sid = lax.axis_index("s")
        wid = cid * 16 + sid
        nk = KBASE + (wid < KREM).astype(jnp.int32)
        lsem = (l0, l1)
        ssem = (s0, s1)

        def start_loads(slot, i):
            lbase = (wid + i * NW) * C
            pltpu.async_copy(dst_h.at[pl.ds(slab * ES + lbase, C)], idx.at[slot], lsem[slot])
            pltpu.async_copy(e_h.at[pl.ds(lbase, C)], buf.at[slot], lsem[slot])

        def wait_loads(slot):
            pltpu.make_async_copy(dst_h.at[pl.ds(0, C)], idx.at[slot], lsem[slot]).wait()
            pltpu.make_async_copy(e_h.at[pl.ds(0, C)], buf.at[slot], lsem[slot]).wait()

        def wait_scatter(slot):
            pltpu.make_async_copy(buf.at[slot], acc.at[idx.at[slot]], ssem[slot]).wait()

        start_loads(0, 0)
        r0 = sid * STRIPE
        pltpu.sync_copy(z_h.at[pl.ds(r0, STRIPE)], acc.at[pl.ds(r0, STRIPE)])

        @pl.when(sid == 15)
        def _():
            pltpu.sync_copy(z_h.at[pl.ds(TAIL0, TAILN)], acc.at[pl.ds(TAIL0, TAILN)])

        plsc.subcore_barrier()

        def body(i, carry):
            def step(cur, nxt):
                wait_loads(cur)

                @pl.when(i + 1 < nk)
                def _():
                    @pl.when(i >= 1)
                    def _():
                        wait_scatter(nxt)

                    start_loads(nxt, i + 1)

                pltpu.async_copy(buf.at[cur], acc.at[idx.at[cur]], ssem[cur],
                                 add=True)

            @pl.when(lax.rem(i, 2) == 0)
            def _():
                step(0, 1)

            @pl.when(lax.rem(i, 2) == 1)
            def _():
                step(1, 0)

            return carry

        lax.fori_loop(0, nk, body, 0)
        wait_scatter(0)
        wait_scatter(1)
        plsc.subcore_barrier()
        pltpu.sync_copy(acc.at[pl.ds(r0, STRIPE)],
                        out_h.at[cid, pl.ds(r0, STRIPE)])

        @pl.when(sid == 15)
        def _():
            pltpu.sync_copy(acc.at[pl.ds(TAIL0, TAILN)],
                            out_h.at[cid, pl.ds(TAIL0, TAILN)])

    return k(e, dst, zeros_nh)


_NB = 1000
_EB = 2000


def _w_spec(shape):
    nd = len(shape)
    return pl.BlockSpec(shape, (lambda i: (0,) * nd))


def _node_encode(x, W1, b1, W2, b2, g, be, Ws, Wd):
    def body(x_r, W1_r, b1_r, W2_r, b2_r, g_r, be_r, Ws_r, Wd_r, h_r, ps_r, pd_r):
        u = jax.nn.silu(jnp.dot(x_r[...], W1_r[...]) + b1_r[...])
        y = jnp.dot(u, W2_r[...]) + b2_r[...]
        h = _ln(y, g_r[...], be_r[...])
        h_r[...] = h
        ps_r[...] = jnp.dot(h, Ws_r[...])
        pd_r[...] = jnp.dot(h, Wd_r[...])

    nblk = pl.BlockSpec((_NB, H), lambda i: (i, 0))
    return pl.pallas_call(
        body,
        grid=(N // _NB,),
        in_specs=[pl.BlockSpec((_NB, x.shape[1]), lambda i: (i, 0)),
                  _w_spec(W1.shape), _w_spec(b1.shape), _w_spec(W2.shape),
                  _w_spec(b2.shape), _w_spec(g.shape), _w_spec(be.shape),
                  _w_spec(Ws.shape), _w_spec(Wd.shape)],
        out_specs=[nblk, nblk, nblk],
        out_shape=[jax.ShapeDtypeStruct((N, H), f32)] * 3,
    )(x, W1, b1, W2, b2, g, be, Ws, Wd)


def _edge_encode(ea, W1, b1, W2, b2, g, be, slab):
    def body(ea_r, W1_r, b1_r, W2_r, b2_r, g_r, be_r, e_r):
        u = jax.nn.silu(jnp.dot(ea_r[...], W1_r[...]) + b1_r[...])
        y = jnp.dot(u, W2_r[...]) + b2_r[...]
        e_r[...] = _ln(y, g_r[...], be_r[...])

    off = slab * (ES // _EB)
    return pl.pallas_call(
        body,
        grid=(ES // _EB,),
        in_specs=[pl.BlockSpec((_EB, ea.shape[1]), lambda i: (i + off, 0)),
                  _w_spec(W1.shape), _w_spec(b1.shape), _w_spec(W2.shape),
                  _w_spec(b2.shape), _w_spec(g.shape), _w_spec(be.shape)],
        out_specs=pl.BlockSpec((_EB, H), lambda i: (i, 0)),
        out_shape=jax.ShapeDtypeStruct((ES, H), f32),
    )(ea, W1, b1, W2, b2, g, be)


def _edge_mlp(e, gsum, We, b1, W2, b2, g, be):
    def body(e_r, gs_r, We_r, b1_r, W2_r, b2_r, g_r, be_r, o_r):
        e0 = e_r[...]
        u = jax.nn.silu(jnp.dot(e0, We_r[...]) + gs_r[...].astype(f32) + b1_r[...])
        y = jnp.dot(u, W2_r[...]) + b2_r[...]
        o_r[...] = e0 + _ln(y, g_r[...], be_r[...])

    eblk = pl.BlockSpec((_EB, H), lambda i: (i, 0))
    return pl.pallas_call(
        body,
        grid=(ES // _EB,),
        in_specs=[eblk, eblk, _w_spec(We.shape), _w_spec(b1.shape),
                  _w_spec(W2.shape), _w_spec(b2.shape), _w_spec(g.shape),
                  _w_spec(be.shape)],
        out_specs=eblk,
        out_shape=jax.ShapeDtypeStruct((ES, H), f32),
    )(e, gsum, We, b1, W2, b2, g, be)


def _node_update(h, parts_list, Wh, Wa, b1, W2, b2, g, be, Ws, Wd):
    def body(h_r, *refs):
        (p_refs, (Wh_r, Wa_r, b1_r, W2_r, b2_r, g_r, be_r, Ws_r, Wd_r),
         (hn_r, ps_r, pd_r)) = refs[:NSLAB], refs[NSLAB:NSLAB + 9], refs[NSLAB + 9:]
        h0 = h_r[...]
        agg = sum(p_r[0] + p_r[1] for p_r in p_refs)
        u = jax.nn.silu(jnp.dot(h0, Wh_r[...]) + jnp.dot(agg, Wa_r[...]) + b1_r[...])
        y = jnp.dot(u, W2_r[...]) + b2_r[...]
        hn = h0 + _ln(y, g_r[...], be_r[...])
        hn_r[...] = hn
        ps_r[...] = jnp.dot(hn, Ws_r[...])
        pd_r[...] = jnp.dot(hn, Wd_r[...])

    nblk = pl.BlockSpec((_NB, H), lambda i: (i, 0))
    pblk = pl.BlockSpec((2, _NB, H), lambda i: (0, i, 0))
    return pl.pallas_call(
        body,
        grid=(N // _NB,),
        in_specs=[nblk] + [pblk] * NSLAB +
                 [_w_spec(Wh.shape), _w_spec(Wa.shape), _w_spec(b1.shape),
                  _w_spec(W2.shape), _w_spec(b2.shape), _w_spec(g.shape),
                  _w_spec(be.shape), _w_spec(Ws.shape), _w_spec(Wd.shape)],
        out_specs=[nblk, nblk, nblk],
        out_shape=[jax.ShapeDtypeStruct((N, H), f32)] * 3,
    )(h, *parts_list, Wh, Wa, b1, W2, b2, g, be, Ws, Wd)


def _node_update_final(h, parts_list, Wh, Wa, b1, W2, b2, g, be, dW1, db1, dW2, db2):
    def body(h_r, *refs):
        (p_refs, (Wh_r, Wa_r, b1_r, W2_r, b2_r, g_r, be_r,
                  dW1_r, db1_r, dW2_r, db2_r), (o_r,)) = (
            refs[:NSLAB], refs[NSLAB:NSLAB + 11], refs[NSLAB + 11:])
        h0 = h_r[...]
        agg = sum(p_r[0] + p_r[1] for p_r in p_refs)
        u = jax.nn.silu(jnp.dot(h0, Wh_r[...]) + jnp.dot(agg, Wa_r[...]) + b1_r[...])
        y = jnp.dot(u, W2_r[...]) + b2_r[...]
        hn = h0 + _ln(y, g_r[...], be_r[...])
        d = jax.nn.silu(jnp.dot(hn, dW1_r[...]) + db1_r[...])
        o_r[...] = jnp.dot(d, dW2_r[...]) + db2_r[...]

    nblk = pl.BlockSpec((_NB, H), lambda i: (i, 0))
    pblk = pl.BlockSpec((2, _NB, H), lambda i: (0, i, 0))
    D_OUT = dW2.shape[1]
    return pl.pallas_call(
        body,
        grid=(N // _NB,),
        in_specs=[nblk] + [pblk] * NSLAB +
                 [_w_spec(Wh.shape), _w_spec(Wa.shape), _w_spec(b1.shape),
                  _w_spec(W2.shape), _w_spec(b2.shape), _w_spec(g.shape),
                  _w_spec(be.shape), _w_spec(dW1.shape), _w_spec(db1.shape),
                  _w_spec(dW2.shape), _w_spec(db2.shape)],
        out_specs=pl.BlockSpec((_NB, D_OUT), lambda i: (i, 0)),
        out_shape=jax.ShapeDtypeStruct((N, D_OUT), f32),
    )(h, *parts_list, Wh, Wa, b1, W2, b2, g, be, dW1, db1, dW2, db2)


def kernel(x, edge_index, edge_attr, en_W1, en_b1, en_W2, en_b2, en_g, en_be, ee_W1, ee_b1, ee_W2, ee_b2, ee_g, ee_be, pe_W1, pe_b1, pe_W2, pe_b2, pe_g, pe_be, pn_W1, pn_b1, pn_W2, pn_b2, pn_g, pn_be, dec_W1, dec_b1, dec_W2, dec_b2):
    src = edge_index[0]
    dst = edge_index[1]
    L = pe_W1.shape[0]
    zeros_nh = jnp.zeros((N, H), f32)

    def row(v):
        return v.reshape(1, -1)

    h, ps, pd = _node_encode(x, en_W1, row(en_b1), en_W2, row(en_b2),
                             row(en_g), row(en_be),
                             pe_W1[0, H:2 * H], pe_W1[0, 2 * H:])
    e = [_edge_encode(edge_attr, ee_W1, row(ee_b1), ee_W2, row(ee_b2),
                      row(ee_g), row(ee_be), s_)
         for s_ in range(NSLAB)]

    for l in range(L):
        parts = [None] * NSLAB
        for s_ in range(NSLAB):
            g = _gather_add(ps, pd, src, dst, s_)
            e[s_] = _edge_mlp(e[s_], g, pe_W1[l, :H], row(pe_b1[l]), pe_W2[l],
                              row(pe_b2[l]), row(pe_g[l]), row(pe_be[l]))
            parts[s_] = _seg_sum(e[s_], dst, zeros_nh, s_)
        if l < L - 1:
            h, ps, pd = _node_update(
                h, parts, pn_W1[l, :H], pn_W1[l, H:], row(pn_b1[l]),
                pn_W2[l], row(pn_b2[l]), row(pn_g[l]), row(pn_be[l]),
                pe_W1[l + 1, H:2 * H], pe_W1[l + 1, 2 * H:])
        else:
            out = _node_update_final(
                h, parts, pn_W1[l, :H], pn_W1[l, H:], row(pn_b1[l]),
                pn_W2[l], row(pn_b2[l]), row(pn_g[l]), row(pn_be[l]),
                dec_W1, row(dec_b1), dec_W2, row(dec_b2))
    return out


# speedup vs baseline: 1.0370x; 1.0097x over previous
import functools

import jax
import jax.numpy as jnp
from jax import lax
from jax.experimental import pallas as pl
from jax.experimental.pallas import tpu as pltpu
from jax.experimental.pallas import tpu_sc as plsc

N = 10000
E = 320000
H = 128
C = 128
NW = 32
NSLAB = 2
ES = E // NSLAB
SCHUNKS = ES // C
KBASE = SCHUNKS // NW
KREM = SCHUNKS % NW
STRIPE = 624
TAIL0 = 16 * STRIPE
TAILN = N - TAIL0

f32 = jnp.float32


def _ln(y, g, b):
    m = jnp.mean(y, axis=-1, keepdims=True)
    v = jnp.mean((y - m) ** 2, axis=-1, keepdims=True)
    return (y - m) * lax.rsqrt(v + 1e-5) * g + b


def _sc_mesh():
    return plsc.VectorSubcoreMesh(core_axis_name="c", subcore_axis_name="s")


def _gather_add(ps, pd, src, dst, slab):
    @functools.partial(
        pl.kernel,
        out_type=jax.ShapeDtypeStruct((ES, H), f32),
        mesh=_sc_mesh(),
        scratch_types=[
            pltpu.VMEM((2, C), jnp.int32),
            pltpu.VMEM((2, C), jnp.int32),
            pltpu.VMEM((2, C, H), f32),
            pltpu.VMEM((2, C, H), f32),
            pltpu.SemaphoreType.DMA,
            pltpu.SemaphoreType.DMA,
            pltpu.SemaphoreType.DMA,
            pltpu.SemaphoreType.DMA,
        ],
    )
    def k(ps_h, pd_h, src_h, dst_h, g_h, idx_s, idx_d, buf_a, buf_b,
          g0, g1, w0, w1):
        cid = lax.axis_index("c")
        sid = lax.axis_index("s")
        wid = cid * 16 + sid
        nk = KBASE + (wid < KREM).astype(jnp.int32)
        gsem = (g0, g1)
        wsem = (w0, w1)

        def start_gathers(slot, i):
            base = (slab * SCHUNKS + wid + i * NW) * C
            ia = pltpu.async_copy(src_h.at[pl.ds(base, C)], idx_s.at[slot], gsem[slot])
            ib = pltpu.async_copy(dst_h.at[pl.ds(base, C)], idx_d.at[slot], gsem[slot])
            ia.wait()
            ib.wait()
            pltpu.async_copy(ps_h.at[idx_s.at[slot]], buf_a.at[slot], gsem[slot])
            pltpu.async_copy(pd_h.at[idx_d.at[slot]], buf_b.at[slot], gsem[slot])

        def wait_gathers(slot):
            pltpu.make_async_copy(ps_h.at[idx_s.at[slot]], buf_a.at[slot], gsem[slot]).wait()
            pltpu.make_async_copy(pd_h.at[idx_d.at[slot]], buf_b.at[slot], gsem[slot]).wait()

        def wait_wb(slot):
            pltpu.make_async_copy(buf_a.at[slot], g_h.at[pl.ds(0, C)], wsem[slot]).wait()

        start_gathers(0, 0)

        def body(i, carry):
            def step(cur, nxt):
                @pl.when(i + 1 < nk)
                def _():
                    @pl.when(i >= 1)
                    def _():
                        wait_wb(nxt)

                    start_gathers(nxt, i + 1)

                wait_gathers(cur)

                def add_row(j2, c2):
                    for r in range(2):
                        j = j2 * 2 + r
                        for t in range(H // 16):
                            sl = pl.ds(t * 16, 16)
                            buf_a[cur, j, sl] = buf_a[cur, j, sl] + buf_b[cur, j, sl]
                    return c2

                lax.fori_loop(0, C // 2, add_row, 0)
                base = (wid + i * NW) * C
                pltpu.async_copy(buf_a.at[cur], g_h.at[pl.ds(base, C)], wsem[cur])

            @pl.when(lax.rem(i, 2) == 0)
            def _():
                step(0, 1)

            @pl.when(lax.rem(i, 2) == 1)
            def _():
                step(1, 0)

            return carry

        lax.fori_loop(0, nk, body, 0)
        wait_wb(0)
        wait_wb(1)

    return k(ps, pd, src, dst)


def _seg_sum(e, dst, zeros_nh, slab):
    @functools.partial(
        pl.kernel,
        out_type=jax.ShapeDtypeStruct((2, N, H), f32),
        mesh=_sc_mesh(),
        scratch_types=[
            pltpu.VMEM((2, C), jnp.int32),
            pltpu.VMEM((2, C, H), f32),
            pltpu.VMEM_SHARED((N, H), f32),
            pltpu.SemaphoreType.DMA,
            pltpu.SemaphoreType.DMA,
            pltpu.SemaphoreType.DMA,
            pltpu.SemaphoreType.DMA,
        ],
    )
    def k(e_h, dst_h, z_h, out_h, idx, buf, acc, l0, l1, s0, s1):
        cid = lax.axis_index("c")
        sid = lax.axis_index("s")
        wid = cid * 16 + sid
        nk = KBASE + (wid < KREM).astype(jnp.int32)
        lsem = (l0, l1)
        ssem = (s0, s1)

        def start_loads(slot, i):
            lbase = (wid + i * NW) * C
            pltpu.async_copy(dst_h.at[pl.ds(slab * ES + lbase, C)], idx.at[slot], lsem[slot])
            pltpu.async_copy(e_h.at[pl.ds(lbase, C)], buf.at[slot], lsem[slot])

        def wait_loads(slot):
            pltpu.make_async_copy(dst_h.at[pl.ds(0, C)], idx.at[slot], lsem[slot]).wait()
            pltpu.make_async_copy(e_h.at[pl.ds(0, C)], buf.at[slot], lsem[slot]).wait()

        def wait_scatter(slot):
            pltpu.make_async_copy(buf.at[slot], acc.at[idx.at[slot]], ssem[slot]).wait()

        start_loads(0, 0)
        r0 = sid * STRIPE
        pltpu.sync_copy(z_h.at[pl.ds(r0, STRIPE)], acc.at[pl.ds(r0, STRIPE)])

        @pl.when(sid == 15)
        def _():
            pltpu.sync_copy(z_h.at[pl.ds(TAIL0, TAILN)], acc.at[pl.ds(TAIL0, TAILN)])

        plsc.subcore_barrier()

        def body(i, carry):
            def step(cur, nxt):
                wait_loads(cur)

                @pl.when(i + 1 < nk)
                def _():
                    @pl.when(i >= 1)
                    def _():
                        wait_scatter(nxt)

                    start_loads(nxt, i + 1)

                pltpu.async_copy(buf.at[cur], acc.at[idx.at[cur]], ssem[cur],
                                 add=True)

            @pl.when(lax.rem(i, 2) == 0)
            def _():
                step(0, 1)

            @pl.when(lax.rem(i, 2) == 1)
            def _():
                step(1, 0)

            return carry

        lax.fori_loop(0, nk, body, 0)
        wait_scatter(0)
        wait_scatter(1)
        plsc.subcore_barrier()
        pltpu.sync_copy(acc.at[pl.ds(r0, STRIPE)],
                        out_h.at[cid, pl.ds(r0, STRIPE)])

        @pl.when(sid == 15)
        def _():
            pltpu.sync_copy(acc.at[pl.ds(TAIL0, TAILN)],
                            out_h.at[cid, pl.ds(TAIL0, TAILN)])

    return k(e, dst, zeros_nh)


_NB = 1000
_EB = 2000


def _w_spec(shape):
    nd = len(shape)
    return pl.BlockSpec(shape, (lambda i: (0,) * nd))


def _node_encode(x, W1, b1, W2, b2, g, be, Ws, Wd):
    def body(x_r, W1_r, b1_r, W2_r, b2_r, g_r, be_r, Ws_r, Wd_r, h_r, ps_r, pd_r):
        u = jax.nn.silu(jnp.dot(x_r[...], W1_r[...]) + b1_r[...])
        y = jnp.dot(u, W2_r[...]) + b2_r[...]
        h = _ln(y, g_r[...], be_r[...])
        h_r[...] = h
        ps_r[...] = jnp.dot(h, Ws_r[...])
        pd_r[...] = jnp.dot(h, Wd_r[...])

    nblk = pl.BlockSpec((_NB, H), lambda i: (i, 0))
    return pl.pallas_call(
        body,
        grid=(N // _NB,),
        in_specs=[pl.BlockSpec((_NB, x.shape[1]), lambda i: (i, 0)),
                  _w_spec(W1.shape), _w_spec(b1.shape), _w_spec(W2.shape),
                  _w_spec(b2.shape), _w_spec(g.shape), _w_spec(be.shape),
                  _w_spec(Ws.shape), _w_spec(Wd.shape)],
        out_specs=[nblk, nblk, nblk],
        out_shape=[jax.ShapeDtypeStruct((N, H), f32)] * 3,
    )(x, W1, b1, W2, b2, g, be, Ws, Wd)


def _edge_encode(ea, W1, b1, W2, b2, g, be, slab):
    def body(ea_r, W1_r, b1_r, W2_r, b2_r, g_r, be_r, e_r):
        u = jax.nn.silu(jnp.dot(ea_r[...], W1_r[...]) + b1_r[...])
        y = jnp.dot(u, W2_r[...]) + b2_r[...]
        e_r[...] = _ln(y, g_r[...], be_r[...])

    off = slab * (ES // _EB)
    return pl.pallas_call(
        body,
        grid=(ES // _EB,),
        in_specs=[pl.BlockSpec((_EB, ea.shape[1]), lambda i: (i + off, 0)),
                  _w_spec(W1.shape), _w_spec(b1.shape), _w_spec(W2.shape),
                  _w_spec(b2.shape), _w_spec(g.shape), _w_spec(be.shape)],
        out_specs=pl.BlockSpec((_EB, H), lambda i: (i, 0)),
        out_shape=jax.ShapeDtypeStruct((ES, H), f32),
    )(ea, W1, b1, W2, b2, g, be)


def _edge_mlp(e, gsum, We, b1, W2, b2, g, be):
    def body(e_r, gs_r, We_r, b1_r, W2_r, b2_r, g_r, be_r, o_r):
        e0 = e_r[...]
        u = jax.nn.silu(jnp.dot(e0, We_r[...]) + gs_r[...].astype(f32) + b1_r[...])
        y = jnp.dot(u, W2_r[...]) + b2_r[...]
        o_r[...] = e0 + _ln(y, g_r[...], be_r[...])

    eblk = pl.BlockSpec((_EB, H), lambda i: (i, 0))
    return pl.pallas_call(
        body,
        grid=(ES // _EB,),
        in_specs=[eblk, eblk, _w_spec(We.shape), _w_spec(b1.shape),
                  _w_spec(W2.shape), _w_spec(b2.shape), _w_spec(g.shape),
                  _w_spec(be.shape)],
        out_specs=eblk,
        out_shape=jax.ShapeDtypeStruct((ES, H), f32),
    )(e, gsum, We, b1, W2, b2, g, be)


def _node_update(h, parts_list, Wh, Wa, b1, W2, b2, g, be, Ws, Wd):
    def body(h_r, *refs):
        (p_refs, (Wh_r, Wa_r, b1_r, W2_r, b2_r, g_r, be_r, Ws_r, Wd_r),
         (hn_r, ps_r, pd_r)) = refs[:NSLAB], refs[NSLAB:NSLAB + 9], refs[NSLAB + 9:]
        h0 = h_r[...]
        agg = sum(p_r[0] + p_r[1] for p_r in p_refs)
        u = jax.nn.silu(jnp.dot(h0, Wh_r[...]) + jnp.dot(agg, Wa_r[...]) + b1_r[...])
        y = jnp.dot(u, W2_r[...]) + b2_r[...]
        hn = h0 + _ln(y, g_r[...], be_r[...])
        hn_r[...] = hn
        ps_r[...] = jnp.dot(hn, Ws_r[...])
        pd_r[...] = jnp.dot(hn, Wd_r[...])

    nblk = pl.BlockSpec((_NB, H), lambda i: (i, 0))
    pblk = pl.BlockSpec((2, _NB, H), lambda i: (0, i, 0))
    return pl.pallas_call(
        body,
        grid=(N // _NB,),
        in_specs=[nblk] + [pblk] * NSLAB +
                 [_w_spec(Wh.shape), _w_spec(Wa.shape), _w_spec(b1.shape),
                  _w_spec(W2.shape), _w_spec(b2.shape), _w_spec(g.shape),
                  _w_spec(be.shape), _w_spec(Ws.shape), _w_spec(Wd.shape)],
        out_specs=[nblk, nblk, nblk],
        out_shape=[jax.ShapeDtypeStruct((N, H), f32)] * 3,
    )(h, *parts_list, Wh, Wa, b1, W2, b2, g, be, Ws, Wd)


def _node_update_final(h, parts_list, Wh, Wa, b1, W2, b2, g, be, dW1, db1, dW2, db2):
    def body(h_r, *refs):
        (p_refs, (Wh_r, Wa_r, b1_r, W2_r, b2_r, g_r, be_r,
                  dW1_r, db1_r, dW2_r, db2_r), (o_r,)) = (
            refs[:NSLAB], refs[NSLAB:NSLAB + 11], refs[NSLAB + 11:])
        h0 = h_r[...]
        agg = sum(p_r[0] + p_r[1] for p_r in p_refs)
        u = jax.nn.silu(jnp.dot(h0, Wh_r[...]) + jnp.dot(agg, Wa_r[...]) + b1_r[...])
        y = jnp.dot(u, W2_r[...]) + b2_r[...]
        hn = h0 + _ln(y, g_r[...], be_r[...])
        d = jax.nn.silu(jnp.dot(hn, dW1_r[...]) + db1_r[...])
        o_r[...] = jnp.dot(d, dW2_r[...]) + db2_r[...]

    nblk = pl.BlockSpec((_NB, H), lambda i: (i, 0))
    pblk = pl.BlockSpec((2, _NB, H), lambda i: (0, i, 0))
    D_OUT = dW2.shape[1]
    return pl.pallas_call(
        body,
        grid=(N // _NB,),
        in_specs=[nblk] + [pblk] * NSLAB +
                 [_w_spec(Wh.shape), _w_spec(Wa.shape), _w_spec(b1.shape),
                  _w_spec(W2.shape), _w_spec(b2.shape), _w_spec(g.shape),
                  _w_spec(be.shape), _w_spec(dW1.shape), _w_spec(db1.shape),
                  _w_spec(dW2.shape), _w_spec(db2.shape)],
        out_specs=pl.BlockSpec((_NB, D_OUT), lambda i: (i, 0)),
        out_shape=jax.ShapeDtypeStruct((N, D_OUT), f32),
    )(h, *parts_list, Wh, Wa, b1, W2, b2, g, be, dW1, db1, dW2, db2)


def kernel(x, edge_index, edge_attr, en_W1, en_b1, en_W2, en_b2, en_g, en_be, ee_W1, ee_b1, ee_W2, ee_b2, ee_g, ee_be, pe_W1, pe_b1, pe_W2, pe_b2, pe_g, pe_be, pn_W1, pn_b1, pn_W2, pn_b2, pn_g, pn_be, dec_W1, dec_b1, dec_W2, dec_b2):
    src = edge_index[0]
    dst = edge_index[1]
    L = pe_W1.shape[0]
    zeros_nh = jnp.zeros((N, H), f32)

    def row(v):
        return v.reshape(1, -1)

    h, ps, pd = _node_encode(x, en_W1, row(en_b1), en_W2, row(en_b2),
                             row(en_g), row(en_be),
                             pe_W1[0, H:2 * H], pe_W1[0, 2 * H:])
    e = [_edge_encode(edge_attr, ee_W1, row(ee_b1), ee_W2, row(ee_b2),
                      row(ee_g), row(ee_be), s_)
         for s_ in range(NSLAB)]

    for l in range(L):
        parts = [None] * NSLAB
        for s_ in range(NSLAB):
            g = _gather_add(ps, pd, src, dst, s_)
            e[s_] = _edge_mlp(e[s_], g, pe_W1[l, :H], row(pe_b1[l]), pe_W2[l],
                              row(pe_b2[l]), row(pe_g[l]), row(pe_be[l]))
            parts[s_] = _seg_sum(e[s_], dst, zeros_nh, s_)
        if l < L - 1:
            h, ps, pd = _node_update(
                h, parts, pn_W1[l, :H], pn_W1[l, H:], row(pn_b1[l]),
                pn_W2[l], row(pn_b2[l]), row(pn_g[l]), row(pn_be[l]),
                pe_W1[l + 1, H:2 * H], pe_W1[l + 1, 2 * H:])
        else:
            out = _node_update_final(
                h, parts, pn_W1[l, :H], pn_W1[l, H:], row(pn_b1[l]),
                pn_W2[l], row(pn_b2[l]), row(pn_g[l]), row(pn_be[l]),
                dec_W1, row(dec_b1), dec_W2, row(dec_b2))
    return out
